# baseline (speedup 1.0000x reference)
import functools

import jax
import jax.numpy as jnp
from jax import lax
from jax.experimental import pallas as pl
from jax.experimental.pallas import tpu as pltpu
from jax.experimental.pallas import tpu_sc as plsc

N = 100000
E = 1600000
M = 128
NHID = 64
MY = 40
K = 10
ALPHA = 0.1

FEAT = 48
NHALF = N // 2
BROWS = 1568
LASTB = N - 63 * BROWS
SUPER = 8 * BROWS
SENT = 1 << 20

EP = 1605632
EROWS = EP // 128
TROWS = EROWS // 16
TROWS2 = EROWS // 32
DEGPAD = 50176

PAD1 = 51200
R1ROWS = PAD1 // 128
TOTAL1 = 32 * 8 * PAD1
PAD2 = 410624
R2ROWS = PAD2 // 128
TOTAL2 = 32 * 8 * PAD2

_mesh = plsc.VectorSubcoreMesh(core_axis_name="c", subcore_axis_name="s")
_scp = pltpu.CompilerParams(needs_layout_passes=False, use_tc_tiling_on_sc=False)


def _deg_body(dst_hbm, deg_hbm, table, dbuf, sem):
    c = lax.axis_index("c")
    s = lax.axis_index("s")
    base = c * NHALF

    def zbody(i, _):
        table[pl.ds(i * 16, 16)] = jnp.zeros((16,), jnp.float32)
        return _
    lax.fori_loop(0, DEGPAD // 16, zbody, 0, unroll=4)

    ones = jnp.ones((16,), jnp.float32)

    def chunk(co, _):
        row0 = pl.multiple_of(s * TROWS + co * 8, 8)
        pltpu.async_copy(dst_hbm.at[pl.ds(row0, 8)], dbuf, sem).wait()

        def body(v, _):
            d = dbuf[v // 8, pl.ds((v % 8) * 16, 16)]
            rel = d - base
            valid = (rel >= 0) & (rel < NHALF)
            idx = jnp.where(valid, rel, NHALF)
            plsc.addupdate_scatter(table, [idx], ones, mask=valid)
            return _

        lax.fori_loop(0, 64, body, 0, unroll=4)
        return _

    lax.fori_loop(0, TROWS // 8, chunk, 0)

    wid = c * 16 + s
    off = pl.multiple_of(wid * DEGPAD, 128)
    pltpu.sync_copy(table, deg_hbm.at[pl.ds(off, DEGPAD)])


_deg_kernel = functools.partial(
    pl.kernel,
    out_type=jax.ShapeDtypeStruct((32 * DEGPAD,), jnp.float32),
    mesh=_mesh,
    scratch_types=[
        pltpu.VMEM((DEGPAD,), jnp.float32),
        pltpu.VMEM((8, 128), jnp.int32),
        pltpu.SemaphoreType.DMA,
    ],
    compiler_params=_scp,
)(_deg_body)


def _radix_vec(sv, kv, dv, stg_s, stg_d, carry, flush):
    ptrs, goffs = carry
    nptrs = []
    ngoffs = []
    for kt in range(8):
        m = kv == kt
        ptr = ptrs[kt]
        plsc.store_compressed(stg_s.at[kt, pl.ds(ptr, 16)], sv, mask=m)
        plsc.store_compressed(stg_d.at[kt, pl.ds(ptr, 16)], dv, mask=m)
        nptr = ptr + jnp.sum(m.astype(jnp.int32))
        full = nptr >= 1024

        @pl.when(full)
        def _():
            flush(kt, goffs[kt])
            ts = stg_s[kt, pl.ds(1024, 16)]
            td = stg_d[kt, pl.ds(1024, 16)]
            stg_s[kt, pl.ds(0, 16)] = ts
            stg_d[kt, pl.ds(0, 16)] = td

        nptrs.append(jnp.where(full, nptr - 1024, nptr))
        ngoffs.append(goffs[kt] + full.astype(jnp.int32))
    return tuple(nptrs), tuple(ngoffs)


def _radix_drain(stg_s, stg_d, carry, flush, cbuf, cnt_hbm, wid, sent_dr):
    iota = lax.iota(jnp.int32, 16)
    zero16 = jnp.zeros((16,), jnp.int32)
    ptrs, goffs = carry
    gfin = []
    for kt in range(8):
        ptr = ptrs[kt]
        stg_s[kt, pl.ds(ptr, 16)] = zero16
        stg_d[kt, pl.ds(ptr, 16)] = sent_dr + iota
        for j in range(64):
            @pl.when(j * 16 >= ptr)
            def _():
                stg_s[kt, pl.ds(j * 16, 16)] = zero16
                stg_d[kt, pl.ds(j * 16, 16)] = sent_dr + iota

        @pl.when(ptr > 0)
        def _():
            flush(kt, goffs[kt])

        gfin.append(goffs[kt] + (ptr > 0).astype(jnp.int32))

    gv = zero16
    for kt in range(8):
        gv = jnp.where(iota == kt, gfin[kt], gv)
    cbuf[pl.ds(0, 16)] = gv
    for j in range(1, 8):
        cbuf[pl.ds(j * 16, 16)] = zero16
    pltpu.sync_copy(cbuf, cnt_hbm.at[pl.ds(pl.multiple_of(wid * 128, 128), 128)])


def _part1_body(src_hbm, dst_hbm, ps_hbm, pd_hbm, cnt_hbm,
                sbuf, dbuf, stg_s, stg_d, cbuf, sem):
    c = lax.axis_index("c")
    s = lax.axis_index("s")
    wid = c * 16 + s

    def flush(kt, goff):
        roff = pl.multiple_of((wid * 8 + kt) * PAD1 + goff * 1024, 128)
        pltpu.sync_copy(stg_s.at[kt, pl.ds(0, 1024)], ps_hbm.at[pl.ds(roff, 1024)])
        pltpu.sync_copy(stg_d.at[kt, pl.ds(0, 1024)], pd_hbm.at[pl.ds(roff, 1024)])

    def chunk(co, carry):
        row0 = pl.multiple_of(wid * TROWS2 + co * 8, 8)
        g0 = pltpu.async_copy(src_hbm.at[pl.ds(row0, 8)], sbuf, sem)
        g1 = pltpu.async_copy(dst_hbm.at[pl.ds(row0, 8)], dbuf, sem)
        g0.wait()
        g1.wait()

        def body(v, cr):
            sv = sbuf[v // 8, pl.ds((v % 8) * 16, 16)]
            dv = dbuf[v // 8, pl.ds((v % 8) * 16, 16)]
            kv = dv // SUPER
            return _radix_vec(sv, kv, dv, stg_s, stg_d, cr, flush)

        return lax.fori_loop(0, 64, body, carry)

    zeros8 = (jnp.int32(0),) * 8
    carry = lax.fori_loop(0, TROWS2 // 8, chunk, (zeros8, zeros8))
    _radix_drain(stg_s, stg_d, carry, flush, cbuf, cnt_hbm, wid,
                 jnp.int32(SENT))


_part1_kernel = functools.partial(
    pl.kernel,
    out_type=[
        jax.ShapeDtypeStruct((TOTAL1,), jnp.int32),
        jax.ShapeDtypeStruct((TOTAL1,), jnp.int32),
        jax.ShapeDtypeStruct((4096,), jnp.int32),
    ],
    mesh=_mesh,
    scratch_types=[
        pltpu.VMEM((8, 128), jnp.int32),
        pltpu.VMEM((8, 128), jnp.int32),
        pltpu.VMEM((8, 1040), jnp.int32),
        pltpu.VMEM((8, 1040), jnp.int32),
        pltpu.VMEM((128,), jnp.int32),
        pltpu.SemaphoreType.DMA,
    ],
    compiler_params=_scp,
)(_part1_body)


def _part2_body(ps1_hbm, pd1_hbm, cnt1_hbm, ps2_hbm, pd2_hbm, cnt2_hbm,
                sbuf, dbuf, stg_s, stg_d, cbuf, cnt1v, sem):
    c = lax.axis_index("c")
    s = lax.axis_index("s")
    w2 = c * 16 + s
    sup = w2 // 4
    iota = lax.iota(jnp.int32, 16)

    def flush(kt, goff):
        roff = pl.multiple_of((w2 * 8 + kt) * PAD2 + goff * 1024, 128)
        pltpu.sync_copy(stg_s.at[kt, pl.ds(0, 1024)], ps2_hbm.at[pl.ds(roff, 1024)])
        pltpu.sync_copy(stg_d.at[kt, pl.ds(0, 1024)], pd2_hbm.at[pl.ds(roff, 1024)])

    zeros8 = (jnp.int32(0),) * 8
    carry = (zeros8, zeros8)
    for i in range(8):
        wsrc = 8 * (w2 - 4 * sup) + i
        coff = pl.multiple_of(wsrc * 128, 128)
        pltpu.sync_copy(cnt1_hbm.at[pl.ds(coff, 128)], cnt1v)
        nb1 = jnp.sum(jnp.where(iota == sup, cnt1v[pl.ds(0, 16)], 0))
        roff1 = (wsrc * 8 + sup) * R1ROWS

        def chunk(g, cr):
            row0 = pl.multiple_of(roff1 + g * 8, 8)
            g0 = pltpu.async_copy(ps1_hbm.at[pl.ds(row0, 8)], sbuf, sem)
            g1 = pltpu.async_copy(pd1_hbm.at[pl.ds(row0, 8)], dbuf, sem)
            g0.wait()
            g1.wait()

            def body(v, crr):
                sv = sbuf[v // 8, pl.ds((v % 8) * 16, 16)]
                dv = dbuf[v // 8, pl.ds((v % 8) * 16, 16)]
                q = dv // BROWS
                kv = q - 8 * sup
                dr = dv - q * BROWS
                return _radix_vec(sv, kv, dr, stg_s, stg_d, crr, flush)

            return lax.fori_loop(0, 64, body, cr)

        carry = lax.fori_loop(0, nb1, chunk, carry)

    _radix_drain(stg_s, stg_d, carry, flush, cbuf, cnt2_hbm, w2,
                 jnp.int32(BROWS))


_part2_kernel = functools.partial(
    pl.kernel,
    out_type=[
        jax.ShapeDtypeStruct((TOTAL2,), jnp.int32),
        jax.ShapeDtypeStruct((TOTAL2,), jnp.int32),
        jax.ShapeDtypeStruct((4096,), jnp.int32),
    ],
    mesh=_mesh,
    scratch_types=[
        pltpu.VMEM((8, 128), jnp.int32),
        pltpu.VMEM((8, 128), jnp.int32),
        pltpu.VMEM((8, 1040), jnp.int32),
        pltpu.VMEM((8, 1040), jnp.int32),
        pltpu.VMEM((128,), jnp.int32),
        pltpu.VMEM((128,), jnp.int32),
        pltpu.SemaphoreType.DMA,
    ],
    compiler_params=_scp,
)(_part2_body)


def _prop_body(zp_hbm, ps2_hbm, pd2_hbm, cnt2_hbm, out_hbm,
               sidx, didx, rows, cntv, table, gsem):
    c = lax.axis_index("c")
    s = lax.axis_index("s")
    iota = lax.iota(jnp.int32, 16)

    for p in range(2):
        b = p * 32 + c * 16 + s
        sup = b // 8
        sub = b - 8 * sup
        off = pl.multiple_of(b * BROWS, 8)

        @pl.when(b < 63)
        def _():
            pltpu.sync_copy(zp_hbm.at[pl.ds(off, BROWS)],
                            table.at[pl.ds(0, BROWS)])

        @pl.when(b == 63)
        def _():
            pltpu.sync_copy(zp_hbm.at[pl.ds(off, LASTB)],
                            table.at[pl.ds(0, LASTB)])

        for i in range(4):
            w2 = 4 * sup + i
            coff = pl.multiple_of(w2 * 128, 128)
            pltpu.sync_copy(cnt2_hbm.at[pl.ds(coff, 128)], cntv.at[i])

        for i in range(4):
            nb = jnp.sum(jnp.where(iota == sub, cntv[i, pl.ds(0, 16)], 0))
            w2 = 4 * sup + i
            roff = (w2 * 8 + sub) * R2ROWS

            def group(g, _):
                row0 = pl.multiple_of(roff + g * 8, 8)
                g0 = pltpu.async_copy(ps2_hbm.at[pl.ds(row0, 8)], sidx, gsem)
                g1 = pltpu.async_copy(pd2_hbm.at[pl.ds(row0, 8)], didx, gsem)
                g0.wait()
                g1.wait()
                descs = []
                for bb in range(8):
                    descs.append(pltpu.async_copy(
                        zp_hbm.at[sidx.at[bb]],
                        rows.at[pl.ds(bb * 128, 128)], gsem))
                for d in descs:
                    d.wait()

                @plsc.parallel_loop(0, 64)
                def _vec(v):
                    d = didx[v // 8, pl.ds((v % 8) * 16, 16)]
                    ev = v * 16 + iota
                    for f in range(FEAT):
                        fv = jnp.full((16,), f, jnp.int32)
                        vals = plsc.load_gather(rows, [ev, fv])
                        plsc.addupdate_scatter(table, [d, fv], vals)

                return _

            lax.fori_loop(0, nb, group, 0)

        @pl.when(b < 63)
        def _():
            pltpu.sync_copy(table.at[pl.ds(0, BROWS)],
                            out_hbm.at[pl.ds(off, BROWS)])

        @pl.when(b == 63)
        def _():
            pltpu.sync_copy(table.at[pl.ds(0, LASTB)],
                            out_hbm.at[pl.ds(off, LASTB)])


_prop_kernel = functools.partial(
    pl.kernel,
    out_type=jax.ShapeDtypeStruct((N, FEAT), jnp.float32),
    mesh=_mesh,
    scratch_types=[
        pltpu.VMEM((8, 128), jnp.int32),
        pltpu.VMEM((8, 128), jnp.int32),
        pltpu.VMEM((1024, FEAT), jnp.float32),
        pltpu.VMEM((4, 128), jnp.int32),
        pltpu.VMEM((BROWS + 16, FEAT), jnp.float32),
        pltpu.SemaphoreType.DMA,
    ],
    compiler_params=_scp,
)(_prop_body)


ROW_BLK = 2000


def _consts_body(x_ref, w1t_ref, b1_ref, w2t_ref, b2_ref, deg_ref,
                 h_ref, z0p_ref, c1_ref, bf_ref):
    d = jnp.sum(deg_ref[...], axis=1, keepdims=True) + 1.0
    dinv = lax.rsqrt(d)
    h = jnp.maximum(x_ref[...] @ w1t_ref[...] + b1_ref[...], 0.0)
    h = h @ w2t_ref[...] + b2_ref[...]
    h48 = jnp.concatenate([h, jnp.zeros((ROW_BLK, FEAT - MY), jnp.float32)],
                          axis=1)
    h_ref[...] = h48
    z0p_ref[...] = dinv * h48
    c1_ref[...] = (1.0 - ALPHA) / d
    bf_ref[...] = (1.0 - ALPHA) * dinv


def _consts(x, W1, b1, W2, b2, deg_raw):
    return pl.pallas_call(
        _consts_body,
        grid=(N // ROW_BLK,),
        in_specs=[
            pl.BlockSpec((ROW_BLK, M), lambda i: (i, 0)),
            pl.BlockSpec((M, NHID), lambda i: (0, 0)),
            pl.BlockSpec((1, NHID), lambda i: (0, 0)),
            pl.BlockSpec((NHID, MY), lambda i: (0, 0)),
            pl.BlockSpec((1, MY), lambda i: (0, 0)),
            pl.BlockSpec((ROW_BLK, 16), lambda i: (i, 0)),
        ],
        out_specs=[
            pl.BlockSpec((ROW_BLK, FEAT), lambda i: (i, 0)),
            pl.BlockSpec((ROW_BLK, FEAT), lambda i: (i, 0)),
            pl.BlockSpec((ROW_BLK, 1), lambda i: (i, 0)),
            pl.BlockSpec((ROW_BLK, 1), lambda i: (i, 0)),
        ],
        out_shape=[
            jax.ShapeDtypeStruct((N, FEAT), jnp.float32),
            jax.ShapeDtypeStruct((N, FEAT), jnp.float32),
            jax.ShapeDtypeStruct((N, 1), jnp.float32),
            jax.ShapeDtypeStruct((N, 1), jnp.float32),
        ],
    )(x, W1.T, b1[None, :], W2.T, b2[None, :], deg_raw)


def _axpb_body(s_ref, a_ref, b_ref, o_ref):
    o_ref[...] = a_ref[...] * s_ref[...] + ALPHA * b_ref[...]


def _axpb(S, a_col, B):
    return pl.pallas_call(
        _axpb_body,
        grid=(N // ROW_BLK,),
        in_specs=[
            pl.BlockSpec((ROW_BLK, FEAT), lambda i: (i, 0)),
            pl.BlockSpec((ROW_BLK, 1), lambda i: (i, 0)),
            pl.BlockSpec((ROW_BLK, FEAT), lambda i: (i, 0)),
        ],
        out_specs=pl.BlockSpec((ROW_BLK, FEAT), lambda i: (i, 0)),
        out_shape=jax.ShapeDtypeStruct((N, FEAT), jnp.float32),
    )(S, a_col, B)


def _axpb_final_body(s_ref, a_ref, b_ref, o_ref):
    o_ref[...] = (a_ref[...] * s_ref[...] + ALPHA * b_ref[...])[:, :MY]


def _axpb_final(S, a_col, B):
    return pl.pallas_call(
        _axpb_final_body,
        grid=(N // ROW_BLK,),
        in_specs=[
            pl.BlockSpec((ROW_BLK, FEAT), lambda i: (i, 0)),
            pl.BlockSpec((ROW_BLK, 1), lambda i: (i, 0)),
            pl.BlockSpec((ROW_BLK, FEAT), lambda i: (i, 0)),
        ],
        out_specs=pl.BlockSpec((ROW_BLK, MY), lambda i: (i, 0)),
        out_shape=jax.ShapeDtypeStruct((N, MY), jnp.float32),
    )(S, a_col, B)


def kernel(x, edge_index, W1, b1, W2, b2):
    src = edge_index[0]
    dst = edge_index[1]
    src2d = jnp.pad(src, (0, EP - E)).reshape(EROWS, 128)
    dst2d = jnp.pad(dst, (0, EP - E), constant_values=SENT).reshape(EROWS, 128)

    ps1, pd1, cnt1 = _part1_kernel(src2d, dst2d)
    ps2f, pd2f, cnt2 = _part2_kernel(ps1.reshape(TOTAL1 // 128, 128),
                                     pd1.reshape(TOTAL1 // 128, 128), cnt1)
    ps2 = ps2f.reshape(TOTAL2 // 128, 128)
    pd2 = pd2f.reshape(TOTAL2 // 128, 128)

    deg_flat = _deg_kernel(dst2d)
    deg_t = (deg_flat.reshape(2, 16, DEGPAD)[:, :, :NHALF]
             .transpose(0, 2, 1).reshape(N, 16))
    h, z0p, c1, bf = _consts(x, W1, b1, W2, b2, deg_t)

    zp = z0p
    for k in range(K):
        S = _prop_kernel(zp, ps2, pd2, cnt2)
        if k < K - 1:
            zp = _axpb(S, c1, z0p)
        else:
            z = _axpb_final(S, bf, h)
    return z


# speedup vs baseline: 2.2304x; 2.2304x over previous
import functools

import jax
import jax.numpy as jnp
from jax import lax
from jax.experimental import pallas as pl
from jax.experimental.pallas import tpu as pltpu
from jax.experimental.pallas import tpu_sc as plsc

N = 100000
E = 1600000
M = 128
NHID = 64
MY = 40
K = 10
ALPHA = 0.1

NHALF = N // 2
QUARTER = N // 4
NTILE = 1568
NTILE_LAST = QUARTER - 15 * NTILE
DEGPAD = 50176

CHUNK = 1024
NCHUNK = 98
EP = 16 * NCHUNK * CHUNK
EROWS = EP // 128
TROWS = EROWS // 16
TRASH = 2048
SROWS = QUARTER + TRASH

TROWS2 = EROWS // 32
PADQ = 51200
RROWS = PADQ // 128
TOTAL = 128 * PADQ

_mesh = plsc.VectorSubcoreMesh(core_axis_name="c", subcore_axis_name="s")


def _adjust_dst(dstbuf, base, co):
    iota = lax.iota(jnp.int32, 16)

    def body(v, _):
        b = v // 8
        j = v % 8
        d = dstbuf[b, pl.ds(j * 16, 16)]
        rel = d - base
        valid = (rel >= 0) & (rel < QUARTER)
        trash = QUARTER + (((co * 64 + v) & 127) * 16) + iota
        dstbuf[b, pl.ds(j * 16, 16)] = jnp.where(valid, rel, trash)
        return _

    lax.fori_loop(0, 64, body, 0, unroll=4)


def _deg_body(dst_hbm, deg_hbm, table, dbuf, sem):
    c = lax.axis_index("c")
    s = lax.axis_index("s")
    base = c * NHALF

    def zbody(i, _):
        table[pl.ds(i * 16, 16)] = jnp.zeros((16,), jnp.float32)
        return _
    lax.fori_loop(0, DEGPAD // 16, zbody, 0, unroll=4)

    ones = jnp.ones((16,), jnp.float32)

    def chunk(co, _):
        row0 = pl.multiple_of(s * TROWS + co * 8, 8)
        pltpu.async_copy(dst_hbm.at[pl.ds(row0, 8)], dbuf, sem).wait()

        def body(v, _):
            b = v // 8
            j = v % 8
            d = dbuf[b, pl.ds(j * 16, 16)]
            rel = d - base
            valid = (rel >= 0) & (rel < NHALF)
            idx = jnp.where(valid, rel, NHALF)
            plsc.addupdate_scatter(table, [idx], ones, mask=valid)
            return _

        lax.fori_loop(0, 64, body, 0, unroll=4)
        return _

    lax.fori_loop(0, NCHUNK, chunk, 0)

    wid = c * 16 + s
    off = pl.multiple_of(wid * DEGPAD, 128)
    pltpu.sync_copy(table, deg_hbm.at[pl.ds(off, DEGPAD)])


_deg_kernel = functools.partial(
    pl.kernel,
    out_type=jax.ShapeDtypeStruct((32 * DEGPAD,), jnp.float32),
    mesh=_mesh,
    scratch_types=[
        pltpu.VMEM((DEGPAD,), jnp.float32),
        pltpu.VMEM((8, 128), jnp.int32),
        pltpu.SemaphoreType.DMA,
    ],
    compiler_params=pltpu.CompilerParams(needs_layout_passes=False),
)(_deg_body)


def _part_body(src_hbm, dst_hbm, psrc_hbm, pdst_hbm, cnt_hbm,
               sbuf, dbuf, stg_s, stg_d, cbuf, sem):
    c = lax.axis_index("c")
    s = lax.axis_index("s")
    wid = c * 16 + s
    iota = lax.iota(jnp.int32, 16)
    zero16 = jnp.zeros((16,), jnp.int32)

    def flush(qt, goff):
        roff = pl.multiple_of((wid * 4 + qt) * PADQ + goff * 1024, 128)
        pltpu.sync_copy(stg_s.at[qt, pl.ds(0, 1024)],
                        psrc_hbm.at[pl.ds(roff, 1024)])
        pltpu.sync_copy(stg_d.at[qt, pl.ds(0, 1024)],
                        pdst_hbm.at[pl.ds(roff, 1024)])

    def vec(v, carry, b):
        ptrs, goffs = carry
        sv = sbuf[b, pl.ds(v * 16, 16)]
        dv = dbuf[b, pl.ds(v * 16, 16)]
        q = dv // QUARTER
        dr = dv - q * QUARTER
        nptrs = []
        ngoffs = []
        for qt in range(4):
            m = q == qt
            ptr = ptrs[qt]
            plsc.store_compressed(stg_s.at[qt, pl.ds(ptr, 16)], sv, mask=m)
            plsc.store_compressed(stg_d.at[qt, pl.ds(ptr, 16)], dr, mask=m)
            nptr = ptr + jnp.sum(m.astype(jnp.int32))
            full = nptr >= 1024

            @pl.when(full)
            def _():
                flush(qt, goffs[qt])
                ts = stg_s[qt, pl.ds(1024, 16)]
                td = stg_d[qt, pl.ds(1024, 16)]
                stg_s[qt, pl.ds(0, 16)] = ts
                stg_d[qt, pl.ds(0, 16)] = td

            nptrs.append(jnp.where(full, nptr - 1024, nptr))
            ngoffs.append(goffs[qt] + full.astype(jnp.int32))
        return tuple(nptrs), tuple(ngoffs)

    def chunk(co, carry):
        row0 = pl.multiple_of(wid * TROWS2 + co * 8, 8)
        g0 = pltpu.async_copy(src_hbm.at[pl.ds(row0, 8)], sbuf, sem)
        g1 = pltpu.async_copy(dst_hbm.at[pl.ds(row0, 8)], dbuf, sem)
        g0.wait()
        g1.wait()

        def body(v, cr):
            return vec(v % 8, cr, v // 8)

        return lax.fori_loop(0, 64, body, carry)

    zeros4 = (jnp.int32(0),) * 4
    ptrs, goffs = lax.fori_loop(0, TROWS2 // 8, chunk, (zeros4, zeros4))

    gfin = []
    for qt in range(4):
        ptr = ptrs[qt]
        sent_d = QUARTER + iota
        stg_s[qt, pl.ds(ptr, 16)] = zero16
        stg_d[qt, pl.ds(ptr, 16)] = sent_d
        for j in range(64):
            @pl.when(j * 16 >= ptr)
            def _():
                stg_s[qt, pl.ds(j * 16, 16)] = zero16
                stg_d[qt, pl.ds(j * 16, 16)] = QUARTER + ((j % 8) * 16) + iota

        @pl.when(ptr > 0)
        def _():
            flush(qt, goffs[qt])

        gfin.append(goffs[qt] + (ptrs[qt] > 0).astype(jnp.int32))

    gv = zero16
    for qt in range(4):
        gv = jnp.where(iota == qt, gfin[qt], gv)
    cbuf[pl.ds(0, 16)] = gv
    for j in range(1, 8):
        cbuf[pl.ds(j * 16, 16)] = zero16
    pltpu.sync_copy(cbuf, cnt_hbm.at[pl.ds(pl.multiple_of(wid * 128, 128), 128)])


_part_kernel = functools.partial(
    pl.kernel,
    out_type=[
        jax.ShapeDtypeStruct((TOTAL,), jnp.int32),
        jax.ShapeDtypeStruct((TOTAL,), jnp.int32),
        jax.ShapeDtypeStruct((4096,), jnp.int32),
    ],
    mesh=_mesh,
    scratch_types=[
        pltpu.VMEM((8, 128), jnp.int32),
        pltpu.VMEM((8, 128), jnp.int32),
        pltpu.VMEM((4, 1040), jnp.int32),
        pltpu.VMEM((4, 1040), jnp.int32),
        pltpu.VMEM((128,), jnp.int32),
        pltpu.SemaphoreType.DMA,
    ],
    compiler_params=pltpu.CompilerParams(needs_layout_passes=False,
                                         use_tc_tiling_on_sc=False),
)(_part_body)


def _node_rows_copy(s, base, copy_one):
    off = pl.multiple_of(base + s * NTILE, 8)
    loc = pl.multiple_of(s * NTILE, 8)
    copy_one(off, loc, NTILE_LAST)

    @pl.when(s < 15)
    def _():
        off2 = pl.multiple_of(base + s * NTILE + NTILE_LAST, 8)
        loc2 = pl.multiple_of(s * NTILE + NTILE_LAST, 8)
        copy_one(off2, loc2, NTILE - NTILE_LAST)


def _prop_body(zp_hbm, psrc_hbm, pdst_hbm, cnt_hbm, out_hbm,
               sidxa, didxa, sidxb, didxb, rowsa, rowsb, cntv, acc,
               gsema, gsemb, ssema, ssemb):
    c = lax.axis_index("c")
    s = lax.axis_index("s")

    iota = lax.iota(jnp.int32, 16)
    for r in range(2):
        w = 2 * s + r
        off = pl.multiple_of(w * 128, 128)
        pltpu.sync_copy(cnt_hbm.at[pl.ds(off, 128)], cntv.at[r])

    def load_idx(row0, si, di, sem):
        g0 = pltpu.async_copy(psrc_hbm.at[pl.ds(row0, 4)], si, sem)
        g1 = pltpu.async_copy(pdst_hbm.at[pl.ds(row0, 4)], di, sem)
        g0.wait()
        g1.wait()

    def fire_gather(si, rw, sem):
        for bb in range(4):
            pltpu.async_copy(zp_hbm.at[si.at[bb]],
                             rw.at[pl.ds(bb * 128, 128)], sem)

    def drain_scatter(di, rw, sem):
        for bb in range(4):
            pltpu.make_async_copy(rw.at[pl.ds(bb * 128, 128)],
                                  acc.at[di.at[bb]], sem).wait()

    for p in range(2):
        qx = 2 * c + p
        base = c * NHALF + p * QUARTER

        def init_one(off, loc, n):
            pltpu.sync_copy(zp_hbm.at[pl.ds(off, n)], acc.at[pl.ds(loc, n)])

        _node_rows_copy(s, base, init_one)
        plsc.subcore_barrier()

        for r in range(2):
            w = 2 * s + r
            nb = jnp.sum(jnp.where(iota == qx, cntv[r, pl.ds(0, 16)], 0))
            roffr = (w * 4 + qx) * RROWS

            @pl.when(nb > 0)
            def _():
                load_idx(pl.multiple_of(roffr, 8), sidxa, didxa, gsema)
                fire_gather(sidxa, rowsa, gsema)

            def group(g, zz):
                @pl.when(g > 0)
                def _():
                    drain_scatter(didxb, rowsb, ssemb)

                row0 = pl.multiple_of(roffr + g * 8 + 4, 4)
                g0 = pltpu.async_copy(psrc_hbm.at[pl.ds(row0, 4)], sidxb, gsemb)
                g1 = pltpu.async_copy(pdst_hbm.at[pl.ds(row0, 4)], didxb, gsemb)
                g0.wait()
                g1.wait()
                for bb in range(4):
                    pltpu.async_copy(zp_hbm.at[sidxb.at[bb]],
                                     rowsb.at[pl.ds(bb * 128, 128)], gsemb)
                for bb in range(4):
                    pltpu.make_async_copy(
                        zp_hbm.at[sidxa.at[bb]],
                        rowsa.at[pl.ds(bb * 128, 128)], gsema).wait()
                for bb in range(4):
                    pltpu.async_copy(rowsa.at[pl.ds(bb * 128, 128)],
                                     acc.at[didxa.at[bb]], ssema, add=True)
                for bb in range(4):
                    pltpu.make_async_copy(
                        zp_hbm.at[sidxb.at[bb]],
                        rowsb.at[pl.ds(bb * 128, 128)], gsemb).wait()
                for bb in range(4):
                    pltpu.async_copy(rowsb.at[pl.ds(bb * 128, 128)],
                                     acc.at[didxb.at[bb]], ssemb, add=True)

                @pl.when(g < nb - 1)
                def _():
                    for bb in range(4):
                        pltpu.make_async_copy(
                            rowsa.at[pl.ds(bb * 128, 128)],
                            acc.at[didxa.at[bb]], ssema).wait()
                    row1 = pl.multiple_of(roffr + g * 8 + 8, 4)
                    ga = pltpu.async_copy(psrc_hbm.at[pl.ds(row1, 4)],
                                          sidxa, gsema)
                    gb = pltpu.async_copy(pdst_hbm.at[pl.ds(row1, 4)],
                                          didxa, gsema)
                    ga.wait()
                    gb.wait()
                    for bb in range(4):
                        pltpu.async_copy(zp_hbm.at[sidxa.at[bb]],
                                         rowsa.at[pl.ds(bb * 128, 128)],
                                         gsema)
                return zz

            lax.fori_loop(0, nb, group, 0)

            @pl.when(nb > 0)
            def _():
                for bb in range(4):
                    pltpu.make_async_copy(rowsa.at[pl.ds(bb * 128, 128)],
                                          acc.at[didxa.at[bb]], ssema).wait()
                for bb in range(4):
                    pltpu.make_async_copy(rowsb.at[pl.ds(bb * 128, 128)],
                                          acc.at[didxb.at[bb]], ssemb).wait()

        plsc.subcore_barrier()

        def flush_one(off, loc, n):
            pltpu.sync_copy(acc.at[pl.ds(loc, n)], out_hbm.at[pl.ds(off, n)])

        _node_rows_copy(s, base, flush_one)
        plsc.subcore_barrier()


_prop_kernel = functools.partial(
    pl.kernel,
    out_type=jax.ShapeDtypeStruct((N, MY), jnp.float32),
    mesh=_mesh,
    scratch_types=[
        pltpu.VMEM((4, 128), jnp.int32),
        pltpu.VMEM((4, 128), jnp.int32),
        pltpu.VMEM((4, 128), jnp.int32),
        pltpu.VMEM((4, 128), jnp.int32),
        pltpu.VMEM((512, MY), jnp.float32),
        pltpu.VMEM((512, MY), jnp.float32),
        pltpu.VMEM((2, 128), jnp.int32),
        pltpu.VMEM_SHARED((SROWS, MY), jnp.float32),
        pltpu.SemaphoreType.DMA,
        pltpu.SemaphoreType.DMA,
        pltpu.SemaphoreType.DMA,
        pltpu.SemaphoreType.DMA,
    ],
    compiler_params=pltpu.CompilerParams(needs_layout_passes=False,
                                         use_tc_tiling_on_sc=False),
)(_prop_body)


ROW_BLK = 2000


def _consts_body(x_ref, w1t_ref, b1_ref, w2t_ref, b2_ref, deg_ref,
                 h_ref, z0p_ref, c1_ref, bf_ref):
    d = jnp.sum(deg_ref[...], axis=1, keepdims=True) + 1.0
    dinv = lax.rsqrt(d)
    h = jnp.maximum(x_ref[...] @ w1t_ref[...] + b1_ref[...], 0.0)
    h = h @ w2t_ref[...] + b2_ref[...]
    h_ref[...] = h
    z0p_ref[...] = dinv * h
    c1_ref[...] = (1.0 - ALPHA) / d
    bf_ref[...] = (1.0 - ALPHA) * dinv


def _consts(x, W1, b1, W2, b2, deg_raw):
    return pl.pallas_call(
        _consts_body,
        grid=(N // ROW_BLK,),
        in_specs=[
            pl.BlockSpec((ROW_BLK, M), lambda i: (i, 0)),
            pl.BlockSpec((M, NHID), lambda i: (0, 0)),
            pl.BlockSpec((1, NHID), lambda i: (0, 0)),
            pl.BlockSpec((NHID, MY), lambda i: (0, 0)),
            pl.BlockSpec((1, MY), lambda i: (0, 0)),
            pl.BlockSpec((ROW_BLK, 16), lambda i: (i, 0)),
        ],
        out_specs=[
            pl.BlockSpec((ROW_BLK, MY), lambda i: (i, 0)),
            pl.BlockSpec((ROW_BLK, MY), lambda i: (i, 0)),
            pl.BlockSpec((ROW_BLK, 1), lambda i: (i, 0)),
            pl.BlockSpec((ROW_BLK, 1), lambda i: (i, 0)),
        ],
        out_shape=[
            jax.ShapeDtypeStruct((N, MY), jnp.float32),
            jax.ShapeDtypeStruct((N, MY), jnp.float32),
            jax.ShapeDtypeStruct((N, 1), jnp.float32),
            jax.ShapeDtypeStruct((N, 1), jnp.float32),
        ],
    )(x, W1.T, b1[None, :], W2.T, b2[None, :], deg_raw)


def _axpb_body(s_ref, a_ref, b_ref, o_ref):
    o_ref[...] = a_ref[...] * s_ref[...] + ALPHA * b_ref[...]


def _axpb(S, a_col, B):
    return pl.pallas_call(
        _axpb_body,
        grid=(N // ROW_BLK,),
        in_specs=[
            pl.BlockSpec((ROW_BLK, MY), lambda i: (i, 0)),
            pl.BlockSpec((ROW_BLK, 1), lambda i: (i, 0)),
            pl.BlockSpec((ROW_BLK, MY), lambda i: (i, 0)),
        ],
        out_specs=pl.BlockSpec((ROW_BLK, MY), lambda i: (i, 0)),
        out_shape=jax.ShapeDtypeStruct((N, MY), jnp.float32),
    )(S, a_col, B)


def kernel(x, edge_index, W1, b1, W2, b2):
    src = edge_index[0]
    dst = edge_index[1]
    src2d = jnp.pad(src, (0, EP - E)).reshape(EROWS, 128)
    dst2d = jnp.pad(dst, (0, EP - E), constant_values=N).reshape(EROWS, 128)

    psrc_f, pdst_f, cnts = _part_kernel(src2d, dst2d)
    psrc = psrc_f.reshape(TOTAL // 128, 128)
    pdst = pdst_f.reshape(TOTAL // 128, 128)

    deg_flat = _deg_kernel(dst2d)
    deg_t = (deg_flat.reshape(2, 16, DEGPAD)[:, :, :NHALF]
             .transpose(0, 2, 1).reshape(N, 16))
    h, z0p, c1, bf = _consts(x, W1, b1, W2, b2, deg_t)

    zp = z0p
    for k in range(K):
        S = _prop_kernel(zp, psrc, pdst, cnts)
        if k < K - 1:
            zp = _axpb(S, c1, z0p)
        else:
            zp = _axpb(S, bf, h)
    return zp


# speedup vs baseline: 3.1604x; 1.4170x over previous
import functools

import jax
import jax.numpy as jnp
from jax import lax
from jax.experimental import pallas as pl
from jax.experimental.pallas import tpu as pltpu
from jax.experimental.pallas import tpu_sc as plsc

N = 100000
E = 1600000
M = 128
NHID = 64
MY = 40
K = 10
ALPHA = 0.1

NHALF = N // 2
QUARTER = N // 4
NTILE = 1568
NTILE_LAST = QUARTER - 15 * NTILE
DEGPAD = 50176

CHUNK = 1024
NCHUNK = 98
EP = 16 * NCHUNK * CHUNK
EROWS = EP // 128
TROWS = EROWS // 16
TRASH = 2048
SROWS = QUARTER + TRASH

_mesh = plsc.VectorSubcoreMesh(core_axis_name="c", subcore_axis_name="s")


def _deg_body(dst_hbm, deg_hbm, table, dbuf, sem):
    c = lax.axis_index("c")
    s = lax.axis_index("s")
    base = c * NHALF

    def zbody(i, _):
        table[pl.ds(i * 16, 16)] = jnp.zeros((16,), jnp.float32)
        return _
    lax.fori_loop(0, DEGPAD // 16, zbody, 0, unroll=4)

    ones = jnp.ones((16,), jnp.float32)

    def chunk(co, _):
        row0 = pl.multiple_of(s * TROWS + co * 8, 8)
        pltpu.async_copy(dst_hbm.at[pl.ds(row0, 8)], dbuf, sem).wait()

        def body(v, _):
            b = v // 8
            j = v % 8
            d = dbuf[b, pl.ds(j * 16, 16)]
            rel = d - base
            valid = (rel >= 0) & (rel < NHALF)
            idx = jnp.where(valid, rel, NHALF)
            plsc.addupdate_scatter(table, [idx], ones, mask=valid)
            return _

        lax.fori_loop(0, 64, body, 0, unroll=4)
        return _

    lax.fori_loop(0, NCHUNK, chunk, 0)

    wid = c * 16 + s
    off = pl.multiple_of(wid * DEGPAD, 128)
    pltpu.sync_copy(table, deg_hbm.at[pl.ds(off, DEGPAD)])


_deg_kernel = functools.partial(
    pl.kernel,
    out_type=jax.ShapeDtypeStruct((32 * DEGPAD,), jnp.float32),
    mesh=_mesh,
    scratch_types=[
        pltpu.VMEM((DEGPAD,), jnp.float32),
        pltpu.VMEM((8, 128), jnp.int32),
        pltpu.SemaphoreType.DMA,
    ],
    compiler_params=pltpu.CompilerParams(needs_layout_passes=False),
)(_deg_body)


def _node_rows_copy(s, base, copy_one):
    off = pl.multiple_of(base + s * NTILE, 8)
    loc = pl.multiple_of(s * NTILE, 8)
    copy_one(off, loc, NTILE_LAST)

    @pl.when(s < 15)
    def _():
        off2 = pl.multiple_of(base + s * NTILE + NTILE_LAST, 8)
        loc2 = pl.multiple_of(s * NTILE + NTILE_LAST, 8)
        copy_one(off2, loc2, NTILE - NTILE_LAST)


def _adjust32(dstbuf, base, salt):
    iota = lax.iota(jnp.int32, 16)

    def body(v, _):
        b = v // 8
        j = v % 8
        d = dstbuf[b, pl.ds(j * 16, 16)]
        rel = d - base
        valid = (rel >= 0) & (rel < QUARTER)
        trash = QUARTER + (((salt * 32 + v) & 127) * 16) + iota
        dstbuf[b, pl.ds(j * 16, 16)] = jnp.where(valid, rel, trash)
        return _

    lax.fori_loop(0, 32, body, 0, unroll=4)


def _prop_body(zp_hbm, src_hbm, dst_hbm, out_hbm,
               sidxa, didxa, sidxb, didxb, rowsa, rowsb, acc,
               gsema, gsemb, ssema, ssemb):
    c = lax.axis_index("c")
    s = lax.axis_index("s")

    def load_idx(row0, si, di, sem):
        g0 = pltpu.async_copy(src_hbm.at[pl.ds(row0, 4)], si, sem)
        g1 = pltpu.async_copy(dst_hbm.at[pl.ds(row0, 4)], di, sem)
        g0.wait()
        g1.wait()

    def fire_gather(si, rw, sem):
        for bb in range(4):
            pltpu.async_copy(zp_hbm.at[si.at[bb]],
                             rw.at[pl.ds(bb * 128, 128)], sem)

    def drain_gather(si, rw, sem):
        for bb in range(4):
            pltpu.make_async_copy(zp_hbm.at[si.at[bb]],
                                  rw.at[pl.ds(bb * 128, 128)], sem).wait()

    def fire_scatter(di, rw, sem):
        for bb in range(4):
            pltpu.async_copy(rw.at[pl.ds(bb * 128, 128)],
                             acc.at[di.at[bb]], sem, add=True)

    def drain_scatter(di, rw, sem):
        for bb in range(4):
            pltpu.make_async_copy(rw.at[pl.ds(bb * 128, 128)],
                                  acc.at[di.at[bb]], sem).wait()

    for p in range(2):
        base = c * NHALF + p * QUARTER

        def init_one(off, loc, n):
            pltpu.sync_copy(zp_hbm.at[pl.ds(off, n)], acc.at[pl.ds(loc, n)])

        _node_rows_copy(s, base, init_one)
        plsc.subcore_barrier()

        row_base = s * TROWS
        load_idx(pl.multiple_of(row_base, 8), sidxa, didxa, gsema)
        _adjust32(didxa, base, 0)
        fire_gather(sidxa, rowsa, gsema)

        def chunk(g, zz):
            @pl.when(g > 0)
            def _():
                drain_scatter(didxb, rowsb, ssemb)

            row0 = pl.multiple_of(row_base + g * 8 + 4, 4)
            load_idx(row0, sidxb, didxb, gsemb)
            _adjust32(didxb, base, 2 * g + 1)
            fire_gather(sidxb, rowsb, gsemb)
            drain_gather(sidxa, rowsa, gsema)
            fire_scatter(didxa, rowsa, ssema)
            drain_gather(sidxb, rowsb, gsemb)
            fire_scatter(didxb, rowsb, ssemb)

            @pl.when(g < NCHUNK - 1)
            def _():
                drain_scatter(didxa, rowsa, ssema)
                row1 = pl.multiple_of(row_base + g * 8 + 8, 8)
                load_idx(row1, sidxa, didxa, gsema)
                _adjust32(didxa, base, 2 * g + 2)
                fire_gather(sidxa, rowsa, gsema)
            return zz

        lax.fori_loop(0, NCHUNK, chunk, 0)
        drain_scatter(didxa, rowsa, ssema)
        drain_scatter(didxb, rowsb, ssemb)

        plsc.subcore_barrier()

        def flush_one(off, loc, n):
            pltpu.sync_copy(acc.at[pl.ds(loc, n)], out_hbm.at[pl.ds(off, n)])

        _node_rows_copy(s, base, flush_one)
        plsc.subcore_barrier()


_prop_kernel = functools.partial(
    pl.kernel,
    out_type=jax.ShapeDtypeStruct((N, MY), jnp.float32),
    mesh=_mesh,
    scratch_types=[
        pltpu.VMEM((4, 128), jnp.int32),
        pltpu.VMEM((4, 128), jnp.int32),
        pltpu.VMEM((4, 128), jnp.int32),
        pltpu.VMEM((4, 128), jnp.int32),
        pltpu.VMEM((512, MY), jnp.float32),
        pltpu.VMEM((512, MY), jnp.float32),
        pltpu.VMEM_SHARED((SROWS, MY), jnp.float32),
        pltpu.SemaphoreType.DMA,
        pltpu.SemaphoreType.DMA,
        pltpu.SemaphoreType.DMA,
        pltpu.SemaphoreType.DMA,
    ],
    compiler_params=pltpu.CompilerParams(needs_layout_passes=False,
                                         use_tc_tiling_on_sc=False),
)(_prop_body)


ROW_BLK = 2000


def _consts_body(x_ref, w1t_ref, b1_ref, w2t_ref, b2_ref, deg_ref,
                 h_ref, z0p_ref, c1_ref, bf_ref):
    d = jnp.sum(deg_ref[...], axis=1, keepdims=True) + 1.0
    dinv = lax.rsqrt(d)
    h = jnp.maximum(x_ref[...] @ w1t_ref[...] + b1_ref[...], 0.0)
    h = h @ w2t_ref[...] + b2_ref[...]
    h_ref[...] = h
    z0p_ref[...] = dinv * h
    c1_ref[...] = (1.0 - ALPHA) / d
    bf_ref[...] = (1.0 - ALPHA) * dinv


def _consts(x, W1, b1, W2, b2, deg_raw):
    return pl.pallas_call(
        _consts_body,
        grid=(N // ROW_BLK,),
        in_specs=[
            pl.BlockSpec((ROW_BLK, M), lambda i: (i, 0)),
            pl.BlockSpec((M, NHID), lambda i: (0, 0)),
            pl.BlockSpec((1, NHID), lambda i: (0, 0)),
            pl.BlockSpec((NHID, MY), lambda i: (0, 0)),
            pl.BlockSpec((1, MY), lambda i: (0, 0)),
            pl.BlockSpec((ROW_BLK, 16), lambda i: (i, 0)),
        ],
        out_specs=[
            pl.BlockSpec((ROW_BLK, MY), lambda i: (i, 0)),
            pl.BlockSpec((ROW_BLK, MY), lambda i: (i, 0)),
            pl.BlockSpec((ROW_BLK, 1), lambda i: (i, 0)),
            pl.BlockSpec((ROW_BLK, 1), lambda i: (i, 0)),
        ],
        out_shape=[
            jax.ShapeDtypeStruct((N, MY), jnp.float32),
            jax.ShapeDtypeStruct((N, MY), jnp.float32),
            jax.ShapeDtypeStruct((N, 1), jnp.float32),
            jax.ShapeDtypeStruct((N, 1), jnp.float32),
        ],
    )(x, W1.T, b1[None, :], W2.T, b2[None, :], deg_raw)


def _axpb_body(s_ref, a_ref, b_ref, o_ref):
    o_ref[...] = a_ref[...] * s_ref[...] + ALPHA * b_ref[...]


def _axpb(S, a_col, B):
    return pl.pallas_call(
        _axpb_body,
        grid=(N // ROW_BLK,),
        in_specs=[
            pl.BlockSpec((ROW_BLK, MY), lambda i: (i, 0)),
            pl.BlockSpec((ROW_BLK, 1), lambda i: (i, 0)),
            pl.BlockSpec((ROW_BLK, MY), lambda i: (i, 0)),
        ],
        out_specs=pl.BlockSpec((ROW_BLK, MY), lambda i: (i, 0)),
        out_shape=jax.ShapeDtypeStruct((N, MY), jnp.float32),
    )(S, a_col, B)


def kernel(x, edge_index, W1, b1, W2, b2):
    src = edge_index[0]
    dst = edge_index[1]
    src2d = jnp.pad(src, (0, EP - E)).reshape(EROWS, 128)
    dst2d = jnp.pad(dst, (0, EP - E), constant_values=N).reshape(EROWS, 128)

    deg_flat = _deg_kernel(dst2d)
    deg_t = (deg_flat.reshape(2, 16, DEGPAD)[:, :, :NHALF]
             .transpose(0, 2, 1).reshape(N, 16))
    h, z0p, c1, bf = _consts(x, W1, b1, W2, b2, deg_t)

    zp = z0p
    for k in range(K):
        S = _prop_kernel(zp, src2d, dst2d)
        if k < K - 1:
            zp = _axpb(S, c1, z0p)
        else:
            zp = _axpb(S, bf, h)
    return zp
